# Initial kernel scaffold; baseline (speedup 1.0000x reference)
import jax
import jax.numpy as jnp
from jax.experimental import pallas as pl


def kernel(avf_out, seq_len, W1, b1, W2, b2, W3, b3):
    raise NotImplementedError("write your pallas kernel here")


# speedup vs baseline: 1.6194x; 1.6194x over previous
import jax
import jax.numpy as jnp
from jax import lax
from jax.experimental import pallas as pl

B, T, D, H = 16, 4096, 128, 512


def _mlp_kernel(x_ref, w1_ref, b1_ref, w2_ref, b2_ref, w3_ref, b3_ref, out_ref):
    wc = lax.dot_general(w3_ref[...], w2_ref[...], (((1,), (0,)), ((), ())),
                         preferred_element_type=jnp.float32)
    c = jnp.sum(w3_ref[...] * b2_ref[...], axis=1, keepdims=True) + b3_ref[...]

    x = x_ref[0]
    h = lax.dot_general(x, w1_ref[...], (((1,), (1,)), ((), ())),
                        preferred_element_type=jnp.float32)
    h = jnp.maximum(h + b1_ref[...], 0.0)
    z = lax.dot_general(wc, h, (((1,), (1,)), ((), ())),
                        preferred_element_type=jnp.float32)
    out_ref[0] = jax.nn.sigmoid(z + c)


def _topk_kernel(logits_ref, len_ref, out_ref):
    v_raw = logits_ref[...]
    L = len_ref[...]
    k = L // 16 + 1
    col = lax.broadcasted_iota(jnp.int32, (B, T), 1)
    v = jnp.where(col < L, v_raw, -1.0)

    lo0 = jnp.zeros((B, 1), jnp.int32)
    hi0 = jnp.full((B, 1), 0x3F800000, jnp.int32)

    def body(_, carry):
        lo, hi = carry
        mid = (lo + hi) >> 1
        t = lax.bitcast_convert_type(mid, jnp.float32)
        cnt = jnp.sum((v >= t).astype(jnp.int32), axis=1, keepdims=True)
        ge = cnt >= k
        return jnp.where(ge, mid, lo), jnp.where(ge, hi, mid)

    lo, _ = lax.fori_loop(0, 31, body, (lo0, hi0))
    xk = lax.bitcast_convert_type(lo, jnp.float32)
    gt = v > xk
    cnt_gt = jnp.sum(gt.astype(jnp.int32), axis=1, keepdims=True)
    sum_gt = jnp.sum(jnp.where(gt, v, 0.0), axis=1, keepdims=True)
    kf = k.astype(jnp.float32)
    out_ref[...] = (sum_gt + (k - cnt_gt).astype(jnp.float32) * xk) / kf


def kernel(avf_out, seq_len, W1, b1, W2, b2, W3, b3):
    b1r = b1.reshape(1, H)
    b2r = b2.reshape(1, 32)
    b3r = b3.reshape(1, 1)
    lens = seq_len.astype(jnp.int32).reshape(B, 1)

    logits3 = pl.pallas_call(
        _mlp_kernel,
        grid=(B,),
        in_specs=[
            pl.BlockSpec((1, T, D), lambda b: (b, 0, 0)),
            pl.BlockSpec((H, D), lambda b: (0, 0)),
            pl.BlockSpec((1, H), lambda b: (0, 0)),
            pl.BlockSpec((32, H), lambda b: (0, 0)),
            pl.BlockSpec((1, 32), lambda b: (0, 0)),
            pl.BlockSpec((1, 32), lambda b: (0, 0)),
            pl.BlockSpec((1, 1), lambda b: (0, 0)),
        ],
        out_specs=pl.BlockSpec((1, 1, T), lambda b: (b, 0, 0)),
        out_shape=jax.ShapeDtypeStruct((B, 1, T), jnp.float32),
    )(avf_out, W1, b1r, W2, b2r, W3, b3r)
    logits = logits3.reshape(B, T)

    res = pl.pallas_call(
        _topk_kernel,
        out_shape=jax.ShapeDtypeStruct((B, 1), jnp.float32),
    )(logits, lens)
    return res.reshape(B)
